# baseline (speedup 1.0000x reference)
import functools

import jax
import jax.numpy as jnp
from jax import lax
from jax.experimental import pallas as pl
from jax.experimental.pallas import tpu as pltpu
from jax.experimental.pallas import tpu_sc as plsc

B, S, D = 4, 8192, 1024
L = 16
CLS = 101
BIG = 2**30

_mesh = plsc.VectorSubcoreMesh(core_axis_name="c", subcore_axis_name="s", num_cores=1)


@functools.partial(
    pl.kernel,
    out_type=jax.ShapeDtypeStruct((B, D), jnp.float32),
    mesh=_mesh,
    scratch_types=[
        pltpu.VMEM((S,), jnp.int32),
    ],
)
def _cls_pool(ids_hbm, hs_hbm, zero_hbm, out_hbm, ids_v):
    cid = lax.axis_index("c")
    sid = lax.axis_index("s")
    wid = sid + cid * 16

    @pl.when(wid < B)
    def _():
        b = wid
        pltpu.sync_copy(ids_hbm.at[b], ids_v)

        iota = lax.iota(jnp.int32, L)
        U = 8

        def step(i, ms):
            base = i * (U * L)
            out = []
            for u in range(U):
                v = ids_v[pl.ds(base + u * L, L)]
                cand = jnp.where(v == CLS, base + u * L + iota, BIG)
                out.append(jnp.minimum(ms[u], cand))
            return tuple(out)

        init = tuple(jnp.full((L,), BIG, jnp.int32) for _ in range(U))
        ms = lax.fori_loop(0, S // (U * L), step, init)
        m = ms[0]
        for u in range(1, U):
            m = jnp.minimum(m, ms[u])
        pos = m[0]
        for lane in range(1, L):
            pos = jnp.minimum(pos, m[lane])
        found = pos < BIG
        row = b * S + jnp.minimum(pos, S - 1)

        @pl.when(found)
        def _():
            pltpu.sync_copy(hs_hbm.at[row], out_hbm.at[b])

        @pl.when(jnp.logical_not(found))
        def _():
            pltpu.sync_copy(zero_hbm, out_hbm.at[b])


@jax.jit
def kernel(hidden_states, input_ids):
    ids = input_ids.astype(jnp.int32)
    hs = hidden_states.reshape(B * S, D)
    zero = jnp.zeros((D,), jnp.float32)
    return _cls_pool(ids, hs, zero)


# speedup vs baseline: 1.0075x; 1.0075x over previous
import functools

import jax
import jax.numpy as jnp
from jax import lax
from jax.experimental import pallas as pl
from jax.experimental.pallas import tpu as pltpu
from jax.experimental.pallas import tpu_sc as plsc

B, S, D = 4, 8192, 1024
L = 16
CLS = 101
BIG = 2**30
C0 = 2048
U = 8

_mesh = plsc.VectorSubcoreMesh(core_axis_name="c", subcore_axis_name="s", num_cores=1)


@functools.partial(
    pl.kernel,
    out_type=jax.ShapeDtypeStruct((B, D), jnp.float32),
    mesh=_mesh,
    scratch_types=[
        pltpu.VMEM((S,), jnp.int32),
        pltpu.SemaphoreType.DMA,
        pltpu.SemaphoreType.DMA,
    ],
)
def _cls_pool(ids_hbm, hs_hbm, zero_hbm, out_hbm, ids_v, sem_a, sem_b):
    cid = lax.axis_index("c")
    sid = lax.axis_index("s")
    wid = sid + cid * 16

    @pl.when(wid < B)
    def _():
        b = wid
        cp_a = pltpu.async_copy(
            ids_hbm.at[b, pl.ds(0, C0)], ids_v.at[pl.ds(0, C0)], sem_a
        )
        cp_b = pltpu.async_copy(
            ids_hbm.at[b, pl.ds(C0, S - C0)], ids_v.at[pl.ds(C0, S - C0)], sem_b
        )

        iota = lax.iota(jnp.int32, L)

        def scan_range(lo_chunk, n_chunks):
            def step(i, ms):
                base = (lo_chunk + i) * (U * L)
                out = []
                for u in range(U):
                    v = ids_v[pl.ds(base + u * L, L)]
                    cand = jnp.where(v == CLS, base + u * L + iota, BIG)
                    out.append(jnp.minimum(ms[u], cand))
                return tuple(out)

            init = tuple(jnp.full((L,), BIG, jnp.int32) for _ in range(U))
            ms = lax.fori_loop(0, n_chunks, step, init)
            m = ms[0]
            for u in range(1, U):
                m = jnp.minimum(m, ms[u])
            pos = m[0]
            for lane in range(1, L):
                pos = jnp.minimum(pos, m[lane])
            return pos

        cp_a.wait()
        pos_a = scan_range(0, C0 // (U * L))
        cp_b.wait()
        pos = lax.cond(
            pos_a < BIG,
            lambda: pos_a,
            lambda: scan_range(C0 // (U * L), (S - C0) // (U * L)),
        )
        found = pos < BIG
        row = b * S + jnp.minimum(pos, S - 1)

        @pl.when(found)
        def _():
            pltpu.sync_copy(hs_hbm.at[row], out_hbm.at[b])

        @pl.when(jnp.logical_not(found))
        def _():
            pltpu.sync_copy(zero_hbm, out_hbm.at[b])


@jax.jit
def kernel(hidden_states, input_ids):
    ids = input_ids.astype(jnp.int32)
    hs = hidden_states.reshape(B * S, D)
    zero = jnp.zeros((D,), jnp.float32)
    return _cls_pool(ids, hs, zero)


# speedup vs baseline: 1.0129x; 1.0053x over previous
import functools

import jax
import jax.numpy as jnp
from jax import lax
from jax.experimental import pallas as pl
from jax.experimental.pallas import tpu as pltpu
from jax.experimental.pallas import tpu_sc as plsc

B, S, D = 4, 8192, 1024
L = 16
CLS = 101
BIG = 2**30

_mesh = plsc.VectorSubcoreMesh(core_axis_name="c", subcore_axis_name="s", num_cores=1)


@functools.partial(
    pl.kernel,
    out_type=jax.ShapeDtypeStruct((B, D), jnp.float32),
    mesh=_mesh,
    scratch_types=[
        pltpu.VMEM((S,), jnp.int32),
        pltpu.VMEM((D,), jnp.float32),
        pltpu.VMEM((D,), jnp.float32),
    ],
)
def _cls_pool(ids_hbm, hs_hbm, out_hbm, ids_v, row_v, out_v):
    cid = lax.axis_index("c")
    sid = lax.axis_index("s")
    wid = sid + cid * 16

    @pl.when(wid < B)
    def _():
        b = wid
        pltpu.sync_copy(ids_hbm.at[b], ids_v)

        iota = lax.iota(jnp.int32, L)
        U = 8

        def step(i, ms):
            base = i * (U * L)
            out = []
            for u in range(U):
                v = ids_v[pl.ds(base + u * L, L)]
                cand = jnp.where(v == CLS, base + u * L + iota, BIG)
                out.append(jnp.minimum(ms[u], cand))
            return tuple(out)

        init = tuple(jnp.full((L,), BIG, jnp.int32) for _ in range(U))
        ms = lax.fori_loop(0, S // (U * L), step, init)
        m = ms[0]
        for u in range(1, U):
            m = jnp.minimum(m, ms[u])
        pos = m[0]
        for lane in range(1, L):
            pos = jnp.minimum(pos, m[lane])
        found = pos < BIG
        row = b * S + jnp.minimum(pos, S - 1)

        pltpu.sync_copy(hs_hbm.at[row], row_v)

        scale = jnp.where(found, jnp.float32(1.0), jnp.float32(0.0))
        for j in range(D // L):
            out_v[pl.ds(j * L, L)] = row_v[pl.ds(j * L, L)] * scale

        pltpu.sync_copy(out_v, out_hbm.at[b])


@jax.jit
def kernel(hidden_states, input_ids):
    ids = input_ids.astype(jnp.int32)
    hs = hidden_states.reshape(B * S, D)
    return _cls_pool(ids, hs)


# speedup vs baseline: 1.0211x; 1.0081x over previous
import functools

import jax
import jax.numpy as jnp
from jax import lax
from jax.experimental import pallas as pl
from jax.experimental.pallas import tpu as pltpu
from jax.experimental.pallas import tpu_sc as plsc

B, S, D = 4, 8192, 1024
L = 16
CLS = 101
BIG = 2**30

_mesh = plsc.VectorSubcoreMesh(core_axis_name="c", subcore_axis_name="s", num_cores=1)


@functools.partial(
    pl.kernel,
    out_type=jax.ShapeDtypeStruct((B, D), jnp.float32),
    mesh=_mesh,
    scratch_types=[
        pltpu.VMEM((S,), jnp.int32),
        pltpu.VMEM((D,), jnp.float32),
        pltpu.VMEM((D,), jnp.float32),
    ],
)
def _cls_pool(ids_hbm, hs_hbm, out_hbm, ids_v, row_v, out_v):
    cid = lax.axis_index("c")
    sid = lax.axis_index("s")
    wid = sid + cid * 16

    @pl.when(wid < B)
    def _():
        b = wid
        pltpu.sync_copy(ids_hbm.at[b], ids_v)

        iota = lax.iota(jnp.int32, L)
        U = 4

        def step(i, ms):
            base = i * (U * L)
            out = []
            for u in range(U):
                v = ids_v[pl.ds(base + u * L, L)]
                cand = jnp.where(v == CLS, base + u * L + iota, BIG)
                out.append(jnp.minimum(ms[u], cand))
            return tuple(out)

        init = tuple(jnp.full((L,), BIG, jnp.int32) for _ in range(U))
        ms = lax.fori_loop(0, S // (U * L), step, init)
        m = ms[0]
        for u in range(1, U):
            m = jnp.minimum(m, ms[u])
        pos = m[0]
        for lane in range(1, L):
            pos = jnp.minimum(pos, m[lane])
        found = pos < BIG
        row = b * S + jnp.minimum(pos, S - 1)

        pltpu.sync_copy(hs_hbm.at[row], row_v)

        scale = jnp.where(found, jnp.float32(1.0), jnp.float32(0.0))
        for j in range(D // L):
            out_v[pl.ds(j * L, L)] = row_v[pl.ds(j * L, L)] * scale

        pltpu.sync_copy(out_v, out_hbm.at[b])


@jax.jit
def kernel(hidden_states, input_ids):
    ids = input_ids.astype(jnp.int32)
    hs = hidden_states.reshape(B * S, D)
    return _cls_pool(ids, hs)
